# baseline (speedup 1.0000x reference)
import functools

import jax
import jax.numpy as jnp
from jax import lax
from jax.experimental import pallas as pl
from jax.experimental.pallas import tpu as pltpu
from jax.experimental.pallas import tpu_sc as plsc

N = 10000
E = 320000
D = 128
EPS = 1e-5

NC = 2
NS = 16
NW = NC * NS

CH = 128
NPAD = 10240
PADV = 10200

DEG_EPT = E // NS
DG_CH = 160
DEG_RPT = NPAD // NS

EPT = E // NW
AG_CH = 80
GRP = 8
RPT = NPAD // NS
RNCH = RPT // CH

_MESH = dict(core_axis_name="c", subcore_axis_name="s")


def _pad_edges(edge_index):
    src, dst = edge_index[0], edge_index[1]
    apad = AG_CH * CH - EPT
    fill = (PADV + (jnp.arange(apad, dtype=jnp.int32) % 32)).astype(jnp.int32)
    s2 = jnp.concatenate(
        [src.reshape(NW, EPT), jnp.zeros((NW, apad), jnp.int32)], axis=1
    ).reshape(NW * AG_CH * CH)
    d2 = jnp.concatenate(
        [dst.reshape(NW, EPT), jnp.broadcast_to(fill, (NW, apad))], axis=1
    ).reshape(NW * AG_CH * CH)
    gpad = DG_CH * CH - DEG_EPT
    gfill = (PADV + (jnp.arange(gpad, dtype=jnp.int32) % 32)).astype(jnp.int32)
    g2 = jnp.stack([src.reshape(NS, DEG_EPT), dst.reshape(NS, DEG_EPT)])
    g3 = jnp.concatenate(
        [g2, jnp.broadcast_to(gfill, (2, NS, gpad))], axis=2
    ).reshape(2, NS, DG_CH, CH)
    return s2, d2, g3


def _rsqrt16(x):
    bits = lax.bitcast_convert_type(x, jnp.int32)
    i = jnp.int32(0x5F3759DF) - lax.shift_right_logical(bits, 1)
    y = lax.bitcast_convert_type(i, jnp.float32)
    for _ in range(3):
        y = y * (jnp.float32(1.5) - jnp.float32(0.5) * x * y * y)
    return y


def _sc_norms(g3, zeros_deg, ones_v):
    mesh = plsc.VectorSubcoreMesh(**_MESH)

    @functools.partial(
        pl.kernel,
        out_type=jax.ShapeDtypeStruct((NC, NPAD), jnp.float32),
        mesh=mesh,
        scratch_types=[
            pltpu.VMEM_SHARED((NPAD,), jnp.float32),
            pltpu.VMEM((CH,), jnp.float32),
            pltpu.VMEM((DG_CH, CH), jnp.int32),
            pltpu.VMEM((DEG_RPT,), jnp.float32),
            pltpu.VMEM((DEG_RPT,), jnp.float32),
            pltpu.SemaphoreType.DMA,
        ],
    )
    def kern(g3_hbm, zeros_hbm, ones_hbm, norms_hbm,
             acc, ones_b, slab, stripe_b, nbuf, sem):
        cid = lax.axis_index("c")
        sid = lax.axis_index("s")
        pltpu.sync_copy(g3_hbm.at[cid, sid], slab)
        pltpu.sync_copy(zeros_hbm, stripe_b)
        pltpu.sync_copy(stripe_b, acc.at[pl.ds(sid * DEG_RPT, DEG_RPT)])
        pltpu.sync_copy(ones_hbm, ones_b)
        plsc.subcore_barrier()

        @pl.loop(0, DG_CH, step=16)
        def _(c0):
            descs = [
                pltpu.async_copy(ones_b, acc.at[slab.at[c0 + j]], sem, add=True)
                for j in range(16)
            ]
            for dsc in descs:
                dsc.wait()

        plsc.subcore_barrier()

        pltpu.sync_copy(acc.at[pl.ds(sid * DEG_RPT, DEG_RPT)], stripe_b)

        def nbody(j, carry):
            d = stripe_b[pl.ds(j * 16, 16)]
            nbuf[pl.ds(j * 16, 16)] = _rsqrt16(jnp.maximum(d, jnp.float32(1.0)))
            return carry

        lax.fori_loop(0, DEG_RPT // 16, nbody, 0)
        pltpu.sync_copy(nbuf, norms_hbm.at[cid, pl.ds(sid * DEG_RPT, DEG_RPT)])

    return kern(g3, zeros_deg, ones_v)


def _sc_aggregate(hn, s_flat, d_flat, zeros_rows):
    mesh = plsc.VectorSubcoreMesh(**_MESH)

    @functools.partial(
        pl.kernel,
        out_type=jax.ShapeDtypeStruct((NC, NPAD, D), jnp.float32),
        mesh=mesh,
        scratch_types=[
            pltpu.VMEM_SHARED((NPAD, D), jnp.float32),
            pltpu.VMEM((GRP * CH,), jnp.int32),
            pltpu.VMEM((GRP * CH,), jnp.int32),
            pltpu.VMEM((CH, D), jnp.float32),
            pltpu.VMEM((CH, D), jnp.float32),
            pltpu.SemaphoreType.DMA,
        ],
    )
    def kern(hn_hbm, s_hbm, d_hbm, zeros_hbm, out_hbm,
             acc, sgrp, dgrp, rows_a, rows_b, gsem):
        cid = lax.axis_index("c")
        sid = lax.axis_index("s")
        wid = cid * NS + sid
        for k in range(RNCH):
            pltpu.sync_copy(zeros_hbm, acc.at[pl.ds(sid * RPT + k * CH, CH)])
        plsc.subcore_barrier()

        base = wid * (AG_CH * CH)

        def body(gq, carry):
            goff = base + gq * (GRP * CH)
            pltpu.sync_copy(s_hbm.at[pl.ds(goff, GRP * CH)], sgrp)
            pltpu.sync_copy(d_hbm.at[pl.ds(goff, GRP * CH)], dgrp)
            for j in range(GRP):
                pltpu.async_copy(
                    hn_hbm.at[sgrp.at[pl.ds(j * CH, CH)]], rows_a, gsem).wait()
                pltpu.sync_copy(rows_a, acc.at[dgrp.at[pl.ds(j * CH, CH)]],
                                add=True)
            return carry

        lax.fori_loop(0, AG_CH // GRP, body, 0)
        plsc.subcore_barrier()

        for k in range(RNCH):
            r0 = sid * RPT + k * CH
            pltpu.sync_copy(acc.at[pl.ds(r0, CH)], out_hbm.at[cid, pl.ds(r0, CH)])

    return kern(hn, s_flat, d_flat, zeros_rows)


def _tc_scale(h, ns):
    def body(h_ref, ns_ref, o_ref):
        o_ref[...] = h_ref[...] * ns_ref[...]

    return pl.pallas_call(
        body, out_shape=jax.ShapeDtypeStruct((N, D), jnp.float32)
    )(h, ns)


def _bn_relu(y, g, bt):
    mu = jnp.mean(y, axis=0, keepdims=True)
    yc = y - mu
    var = jnp.mean(yc * yc, axis=0, keepdims=True)
    z = g * (yc * lax.rsqrt(var + EPS)) + bt
    return jnp.maximum(z, 0.0)


def _tc_dense_mid(parts, nd, ns, W, b, g, bt):
    def body(p_ref, nd_ref, ns_ref, W_ref, b_ref, g_ref, bt_ref, o_ref):
        x = (p_ref[0, :N] + p_ref[1, :N]) * nd_ref[...]
        y = jnp.dot(x, W_ref[...], preferred_element_type=jnp.float32) + b_ref[...]
        z = _bn_relu(y, g_ref[...], bt_ref[...])
        o_ref[...] = z * ns_ref[...]

    return pl.pallas_call(
        body, out_shape=jax.ShapeDtypeStruct((N, D), jnp.float32)
    )(parts, nd, ns, W, b, g, bt)


def _tc_dense_last(parts, nd, W, b, g, bt, W_fc, b_fc):
    def body(p_ref, nd_ref, W_ref, b_ref, g_ref, bt_ref, Wf_ref, bf_ref, o_ref):
        x = (p_ref[0, :N] + p_ref[1, :N]) * nd_ref[...]
        y = jnp.dot(x, W_ref[...], preferred_element_type=jnp.float32) + b_ref[...]
        z = _bn_relu(y, g_ref[...], bt_ref[...])
        o_ref[...] = (
            jnp.dot(z, Wf_ref[...], preferred_element_type=jnp.float32) + bf_ref[...]
        )

    return pl.pallas_call(
        body, out_shape=jax.ShapeDtypeStruct((N, D), jnp.float32)
    )(parts, nd, W, b, g, bt, W_fc, b_fc)


def kernel(h, edge_index, W0, b0, gamma0, beta0, W1, b1, gamma1, beta1,
           W2, b2, gamma2, beta2, W_fc, b_fc):
    zeros_deg = jnp.zeros((DEG_RPT,), jnp.float32)
    ones_v = jnp.ones((CH,), jnp.float32)
    zeros_rows = jnp.zeros((CH, D), jnp.float32)

    s_flat, d_flat, g3 = _pad_edges(edge_index)
    norms = _sc_norms(g3, zeros_deg, ones_v)
    ns = norms[0, :N].reshape(N, 1)
    nd = norms[1, :N].reshape(N, 1)

    hn = _tc_scale(h, ns)
    for W, b, g, bt in [(W0, b0, gamma0, beta0), (W1, b1, gamma1, beta1)]:
        parts = _sc_aggregate(hn, s_flat, d_flat, zeros_rows)
        hn = _tc_dense_mid(parts, nd, ns, W, b.reshape(1, D), g.reshape(1, D),
                           bt.reshape(1, D))
    parts = _sc_aggregate(hn, s_flat, d_flat, zeros_rows)
    out = _tc_dense_last(parts, nd, W2, b2.reshape(1, D), gamma2.reshape(1, D),
                         beta2.reshape(1, D), W_fc, b_fc.reshape(1, D))
    return out


# speedup vs baseline: 1.0225x; 1.0225x over previous
import functools

import jax
import jax.numpy as jnp
from jax import lax
from jax.experimental import pallas as pl
from jax.experimental.pallas import tpu as pltpu
from jax.experimental.pallas import tpu_sc as plsc

N = 10000
E = 320000
D = 128
EPS = 1e-5

NC = 2
NS = 16
NW = NC * NS

CH = 128
NPAD = 10240
PADV = 10200

DEG_EPT = E // NS
DG_CH = 160
DEG_RPT = NPAD // NS

EPT = E // NW
AG_CH = 80
GRP = 8
RPT = NPAD // NS
RNCH = RPT // CH

_MESH = dict(core_axis_name="c", subcore_axis_name="s")


def _pad_edges(edge_index):
    src, dst = edge_index[0], edge_index[1]
    apad = AG_CH * CH - EPT
    fill = (PADV + (jnp.arange(apad, dtype=jnp.int32) % 32)).astype(jnp.int32)
    s2 = jnp.concatenate(
        [src.reshape(NW, EPT), jnp.zeros((NW, apad), jnp.int32)], axis=1
    ).reshape(NW * AG_CH * CH)
    d2 = jnp.concatenate(
        [dst.reshape(NW, EPT), jnp.broadcast_to(fill, (NW, apad))], axis=1
    ).reshape(NW * AG_CH * CH)
    gpad = DG_CH * CH - DEG_EPT
    gfill = (PADV + (jnp.arange(gpad, dtype=jnp.int32) % 32)).astype(jnp.int32)
    g2 = jnp.stack([src.reshape(NS, DEG_EPT), dst.reshape(NS, DEG_EPT)])
    g3 = jnp.concatenate(
        [g2, jnp.broadcast_to(gfill, (2, NS, gpad))], axis=2
    ).reshape(2, NS, DG_CH, CH)
    return s2, d2, g3


def _rsqrt16(x):
    bits = lax.bitcast_convert_type(x, jnp.int32)
    i = jnp.int32(0x5F3759DF) - lax.shift_right_logical(bits, 1)
    y = lax.bitcast_convert_type(i, jnp.float32)
    for _ in range(3):
        y = y * (jnp.float32(1.5) - jnp.float32(0.5) * x * y * y)
    return y


def _sc_norms(g3, zeros_deg, ones_v):
    mesh = plsc.VectorSubcoreMesh(**_MESH)

    @functools.partial(
        pl.kernel,
        out_type=jax.ShapeDtypeStruct((NC, NPAD), jnp.float32),
        mesh=mesh,
        scratch_types=[
            pltpu.VMEM_SHARED((NPAD,), jnp.float32),
            pltpu.VMEM((CH,), jnp.float32),
            pltpu.VMEM((DG_CH, CH), jnp.int32),
            pltpu.VMEM((DEG_RPT,), jnp.float32),
            pltpu.VMEM((DEG_RPT,), jnp.float32),
            pltpu.SemaphoreType.DMA,
        ],
    )
    def kern(g3_hbm, zeros_hbm, ones_hbm, norms_hbm,
             acc, ones_b, slab, stripe_b, nbuf, sem):
        cid = lax.axis_index("c")
        sid = lax.axis_index("s")
        pltpu.sync_copy(g3_hbm.at[cid, sid], slab)
        pltpu.sync_copy(zeros_hbm, stripe_b)
        pltpu.sync_copy(stripe_b, acc.at[pl.ds(sid * DEG_RPT, DEG_RPT)])
        pltpu.sync_copy(ones_hbm, ones_b)
        plsc.subcore_barrier()

        @pl.loop(0, DG_CH, step=16)
        def _(c0):
            descs = [
                pltpu.async_copy(ones_b, acc.at[slab.at[c0 + j]], sem, add=True)
                for j in range(16)
            ]
            for dsc in descs:
                dsc.wait()

        plsc.subcore_barrier()

        pltpu.sync_copy(acc.at[pl.ds(sid * DEG_RPT, DEG_RPT)], stripe_b)

        def nbody(j, carry):
            d = stripe_b[pl.ds(j * 16, 16)]
            nbuf[pl.ds(j * 16, 16)] = _rsqrt16(jnp.maximum(d, jnp.float32(1.0)))
            return carry

        lax.fori_loop(0, DEG_RPT // 16, nbody, 0)
        pltpu.sync_copy(nbuf, norms_hbm.at[cid, pl.ds(sid * DEG_RPT, DEG_RPT)])

    return kern(g3, zeros_deg, ones_v)


def _sc_aggregate(hn, s_flat, d_flat, zeros_rows):
    mesh = plsc.VectorSubcoreMesh(**_MESH)

    @functools.partial(
        pl.kernel,
        out_type=jax.ShapeDtypeStruct((NC, NPAD, D), jnp.float32),
        mesh=mesh,
        scratch_types=[
            pltpu.VMEM_SHARED((NPAD, D), jnp.float32),
            pltpu.VMEM((CH,), jnp.int32),
            pltpu.VMEM((CH,), jnp.int32),
            pltpu.VMEM((CH,), jnp.int32),
            pltpu.VMEM((CH,), jnp.int32),
            pltpu.VMEM((CH, D), jnp.float32),
            pltpu.VMEM((CH, D), jnp.float32),
            pltpu.SemaphoreType.DMA,
        ],
    )
    def kern(hn_hbm, s_hbm, d_hbm, zeros_hbm, out_hbm,
             acc, sidx_a, didx_a, sidx_b, didx_b, rows_a, rows_b, gsem):
        cid = lax.axis_index("c")
        sid = lax.axis_index("s")
        wid = cid * NS + sid
        for k in range(RNCH):
            pltpu.sync_copy(zeros_hbm, acc.at[pl.ds(sid * RPT + k * CH, CH)])
        plsc.subcore_barrier()

        base = wid * (AG_CH * CH)

        def body(cc, carry):
            off = base + cc * (2 * CH)
            pltpu.sync_copy(s_hbm.at[pl.ds(off, CH)], sidx_a)
            pltpu.sync_copy(d_hbm.at[pl.ds(off, CH)], didx_a)
            ga = pltpu.async_copy(hn_hbm.at[sidx_a], rows_a, gsem)
            pltpu.sync_copy(s_hbm.at[pl.ds(off + CH, CH)], sidx_b)
            pltpu.sync_copy(d_hbm.at[pl.ds(off + CH, CH)], didx_b)
            gb = pltpu.async_copy(hn_hbm.at[sidx_b], rows_b, gsem)
            ga.wait()
            pltpu.sync_copy(rows_a, acc.at[didx_a], add=True)
            gb.wait()
            pltpu.sync_copy(rows_b, acc.at[didx_b], add=True)
            return carry

        lax.fori_loop(0, AG_CH // 2, body, 0)
        plsc.subcore_barrier()

        for k in range(RNCH):
            r0 = sid * RPT + k * CH
            pltpu.sync_copy(acc.at[pl.ds(r0, CH)], out_hbm.at[cid, pl.ds(r0, CH)])

    return kern(hn, s_flat, d_flat, zeros_rows)


def _tc_scale(h, ns):
    def body(h_ref, ns_ref, o_ref):
        o_ref[...] = h_ref[...] * ns_ref[...]

    return pl.pallas_call(
        body, out_shape=jax.ShapeDtypeStruct((N, D), jnp.float32)
    )(h, ns)


def _bn_relu(y, g, bt):
    mu = jnp.mean(y, axis=0, keepdims=True)
    yc = y - mu
    var = jnp.mean(yc * yc, axis=0, keepdims=True)
    z = g * (yc * lax.rsqrt(var + EPS)) + bt
    return jnp.maximum(z, 0.0)


def _tc_dense_mid(parts, nd, ns, W, b, g, bt):
    def body(p_ref, nd_ref, ns_ref, W_ref, b_ref, g_ref, bt_ref, o_ref):
        x = (p_ref[0, :N] + p_ref[1, :N]) * nd_ref[...]
        y = jnp.dot(x, W_ref[...], preferred_element_type=jnp.float32) + b_ref[...]
        z = _bn_relu(y, g_ref[...], bt_ref[...])
        o_ref[...] = z * ns_ref[...]

    return pl.pallas_call(
        body, out_shape=jax.ShapeDtypeStruct((N, D), jnp.float32)
    )(parts, nd, ns, W, b, g, bt)


def _tc_dense_last(parts, nd, W, b, g, bt, W_fc, b_fc):
    def body(p_ref, nd_ref, W_ref, b_ref, g_ref, bt_ref, Wf_ref, bf_ref, o_ref):
        x = (p_ref[0, :N] + p_ref[1, :N]) * nd_ref[...]
        y = jnp.dot(x, W_ref[...], preferred_element_type=jnp.float32) + b_ref[...]
        z = _bn_relu(y, g_ref[...], bt_ref[...])
        o_ref[...] = (
            jnp.dot(z, Wf_ref[...], preferred_element_type=jnp.float32) + bf_ref[...]
        )

    return pl.pallas_call(
        body, out_shape=jax.ShapeDtypeStruct((N, D), jnp.float32)
    )(parts, nd, W, b, g, bt, W_fc, b_fc)


def kernel(h, edge_index, W0, b0, gamma0, beta0, W1, b1, gamma1, beta1,
           W2, b2, gamma2, beta2, W_fc, b_fc):
    zeros_deg = jnp.zeros((DEG_RPT,), jnp.float32)
    ones_v = jnp.ones((CH,), jnp.float32)
    zeros_rows = jnp.zeros((CH, D), jnp.float32)

    s_flat, d_flat, g3 = _pad_edges(edge_index)
    norms = _sc_norms(g3, zeros_deg, ones_v)
    ns = norms[0, :N].reshape(N, 1)
    nd = norms[1, :N].reshape(N, 1)

    hn = _tc_scale(h, ns)
    for W, b, g, bt in [(W0, b0, gamma0, beta0), (W1, b1, gamma1, beta1)]:
        parts = _sc_aggregate(hn, s_flat, d_flat, zeros_rows)
        hn = _tc_dense_mid(parts, nd, ns, W, b.reshape(1, D), g.reshape(1, D),
                           bt.reshape(1, D))
    parts = _sc_aggregate(hn, s_flat, d_flat, zeros_rows)
    out = _tc_dense_last(parts, nd, W2, b2.reshape(1, D), gamma2.reshape(1, D),
                         beta2.reshape(1, D), W_fc, b_fc.reshape(1, D))
    return out


# speedup vs baseline: 1.8474x; 1.8067x over previous
import functools

import jax
import jax.numpy as jnp
from jax import lax
from jax.experimental import pallas as pl
from jax.experimental.pallas import tpu as pltpu
from jax.experimental.pallas import tpu_sc as plsc

N = 10000
E = 320000
D = 128
EPS = 1e-5

NC = 2
NS = 16
NW = NC * NS

CH = 128
NPAD = 10240
PADV = 10200

DEG_EPT = E // NS
DG_CH = 160
DEG_RPT = NPAD // NS

DH = D // NC
SLB = 16
RPT = NPAD // NS
RNCH = RPT // CH

_MESH = dict(core_axis_name="c", subcore_axis_name="s")


def _pad_edges(edge_index):
    src, dst = edge_index[0], edge_index[1]
    gpad = DG_CH * CH - DEG_EPT
    gfill = (PADV + (jnp.arange(gpad, dtype=jnp.int32) % 32)).astype(jnp.int32)
    g2 = jnp.stack([src.reshape(NS, DEG_EPT), dst.reshape(NS, DEG_EPT)])
    g3 = jnp.concatenate(
        [g2, jnp.broadcast_to(gfill, (2, NS, gpad))], axis=2
    ).reshape(2, NS, DG_CH, CH)
    return g3


def _rsqrt16(x):
    bits = lax.bitcast_convert_type(x, jnp.int32)
    i = jnp.int32(0x5F3759DF) - lax.shift_right_logical(bits, 1)
    y = lax.bitcast_convert_type(i, jnp.float32)
    for _ in range(3):
        y = y * (jnp.float32(1.5) - jnp.float32(0.5) * x * y * y)
    return y


def _sc_norms(g3, zeros_deg, ones_v):
    mesh = plsc.VectorSubcoreMesh(**_MESH)

    @functools.partial(
        pl.kernel,
        out_type=jax.ShapeDtypeStruct((NC, NPAD), jnp.float32),
        mesh=mesh,
        scratch_types=[
            pltpu.VMEM_SHARED((NPAD,), jnp.float32),
            pltpu.VMEM((CH,), jnp.float32),
            pltpu.VMEM((DG_CH, CH), jnp.int32),
            pltpu.VMEM((DEG_RPT,), jnp.float32),
            pltpu.VMEM((DEG_RPT,), jnp.float32),
            pltpu.SemaphoreType.DMA,
        ],
    )
    def kern(g3_hbm, zeros_hbm, ones_hbm, norms_hbm,
             acc, ones_b, slab, stripe_b, nbuf, sem):
        cid = lax.axis_index("c")
        sid = lax.axis_index("s")
        pltpu.sync_copy(g3_hbm.at[cid, sid], slab)
        pltpu.sync_copy(zeros_hbm, stripe_b)
        pltpu.sync_copy(stripe_b, acc.at[pl.ds(sid * DEG_RPT, DEG_RPT)])
        pltpu.sync_copy(ones_hbm, ones_b)
        plsc.subcore_barrier()

        @pl.loop(0, DG_CH, step=16)
        def _(c0):
            descs = [
                pltpu.async_copy(ones_b, acc.at[slab.at[c0 + j]], sem, add=True)
                for j in range(16)
            ]
            for dsc in descs:
                dsc.wait()

        plsc.subcore_barrier()

        pltpu.sync_copy(acc.at[pl.ds(sid * DEG_RPT, DEG_RPT)], stripe_b)

        def nbody(j, carry):
            d = stripe_b[pl.ds(j * 16, 16)]
            nbuf[pl.ds(j * 16, 16)] = _rsqrt16(jnp.maximum(d, jnp.float32(1.0)))
            return carry

        lax.fori_loop(0, DEG_RPT // 16, nbody, 0)
        pltpu.sync_copy(nbuf, norms_hbm.at[cid, pl.ds(sid * DEG_RPT, DEG_RPT)])

    return kern(g3, zeros_deg, ones_v)


def _sc_aggregate(hn2, g3, zeros_rows):
    mesh = plsc.VectorSubcoreMesh(**_MESH)

    @functools.partial(
        pl.kernel,
        out_type=jax.ShapeDtypeStruct((NC, NPAD, DH), jnp.float32),
        mesh=mesh,
        scratch_types=[
            pltpu.VMEM_SHARED((NPAD, DH), jnp.float32),
            pltpu.VMEM_SHARED((NPAD, DH), jnp.float32),
            pltpu.VMEM((SLB, CH), jnp.int32),
            pltpu.VMEM((SLB, CH), jnp.int32),
            pltpu.VMEM((CH, DH), jnp.float32),
            pltpu.SemaphoreType.DMA,
        ],
    )
    def kern(hn_hbm, g3_hbm, zeros_hbm, out_hbm,
             hsp, acc, sslab, dslab, rows, gsem):
        cid = lax.axis_index("c")
        sid = lax.axis_index("s")
        r0 = sid * RPT
        pltpu.sync_copy(hn_hbm.at[cid, pl.ds(r0, RPT)], hsp.at[pl.ds(r0, RPT)])
        for k in range(RNCH):
            pltpu.sync_copy(zeros_hbm, acc.at[pl.ds(r0 + k * CH, CH)])
        plsc.subcore_barrier()

        @pl.loop(0, DG_CH, step=SLB)
        def _(c0):
            pltpu.sync_copy(g3_hbm.at[0, sid, pl.ds(c0, SLB)], sslab)
            pltpu.sync_copy(g3_hbm.at[1, sid, pl.ds(c0, SLB)], dslab)
            for j in range(SLB):
                pltpu.async_copy(hsp.at[sslab.at[j]], rows, gsem).wait()
                pltpu.sync_copy(rows, acc.at[dslab.at[j]], add=True)

        plsc.subcore_barrier()

        for k in range(RNCH):
            rk = r0 + k * CH
            pltpu.sync_copy(acc.at[pl.ds(rk, CH)], out_hbm.at[cid, pl.ds(rk, CH)])

    return kern(hn2, g3, zeros_rows)


def _tc_scale(h, ns):
    def body(h_ref, ns_ref, o_ref):
        hn = h_ref[...] * ns_ref[...]
        o_ref[0, :N] = hn[:, :DH]
        o_ref[1, :N] = hn[:, DH:]

    return pl.pallas_call(
        body, out_shape=jax.ShapeDtypeStruct((2, NPAD, DH), jnp.float32)
    )(h, ns)


def _bn_relu(y, g, bt):
    mu = jnp.mean(y, axis=0, keepdims=True)
    yc = y - mu
    var = jnp.mean(yc * yc, axis=0, keepdims=True)
    z = g * (yc * lax.rsqrt(var + EPS)) + bt
    return jnp.maximum(z, 0.0)


def _tc_dense_mid(parts, nd, ns, W, b, g, bt):
    def body(p_ref, nd_ref, ns_ref, W_ref, b_ref, g_ref, bt_ref, o_ref):
        x = jnp.concatenate([p_ref[0, :N], p_ref[1, :N]], axis=1) * nd_ref[...]
        y = jnp.dot(x, W_ref[...], preferred_element_type=jnp.float32) + b_ref[...]
        z = _bn_relu(y, g_ref[...], bt_ref[...])
        zns = z * ns_ref[...]
        o_ref[0, :N] = zns[:, :DH]
        o_ref[1, :N] = zns[:, DH:]

    return pl.pallas_call(
        body, out_shape=jax.ShapeDtypeStruct((2, NPAD, DH), jnp.float32)
    )(parts, nd, ns, W, b, g, bt)


def _tc_dense_last(parts, nd, W, b, g, bt, W_fc, b_fc):
    def body(p_ref, nd_ref, W_ref, b_ref, g_ref, bt_ref, Wf_ref, bf_ref, o_ref):
        x = jnp.concatenate([p_ref[0, :N], p_ref[1, :N]], axis=1) * nd_ref[...]
        y = jnp.dot(x, W_ref[...], preferred_element_type=jnp.float32) + b_ref[...]
        z = _bn_relu(y, g_ref[...], bt_ref[...])
        o_ref[...] = (
            jnp.dot(z, Wf_ref[...], preferred_element_type=jnp.float32) + bf_ref[...]
        )

    return pl.pallas_call(
        body, out_shape=jax.ShapeDtypeStruct((N, D), jnp.float32)
    )(parts, nd, W, b, g, bt, W_fc, b_fc)


def kernel(h, edge_index, W0, b0, gamma0, beta0, W1, b1, gamma1, beta1,
           W2, b2, gamma2, beta2, W_fc, b_fc):
    zeros_deg = jnp.zeros((DEG_RPT,), jnp.float32)
    ones_v = jnp.ones((CH,), jnp.float32)
    zeros_rows = jnp.zeros((CH, DH), jnp.float32)

    g3 = _pad_edges(edge_index)
    norms = _sc_norms(g3, zeros_deg, ones_v)
    ns = norms[0, :N].reshape(N, 1)
    nd = norms[1, :N].reshape(N, 1)

    hn = _tc_scale(h, ns)
    for W, b, g, bt in [(W0, b0, gamma0, beta0), (W1, b1, gamma1, beta1)]:
        parts = _sc_aggregate(hn, g3, zeros_rows)
        hn = _tc_dense_mid(parts, nd, ns, W, b.reshape(1, D), g.reshape(1, D),
                           bt.reshape(1, D))
    parts = _sc_aggregate(hn, g3, zeros_rows)
    out = _tc_dense_last(parts, nd, W2, b2.reshape(1, D), gamma2.reshape(1, D),
                         beta2.reshape(1, D), W_fc, b_fc.reshape(1, D))
    return out
